# baseline (speedup 1.0000x reference)
import functools

import jax
import jax.numpy as jnp
from jax import lax
from jax.experimental import pallas as pl
from jax.experimental.pallas import tpu as pltpu
from jax.experimental.pallas import tpu_sc as plsc

_N = 10000
_E = 320000
_D = 128
_H = 64
_C = 64

_NCORES = 2
_NSUB = 16
_NTILES = _NCORES * _NSUB
_CHUNK = 128
_CPT = 80
_EPAD = _NTILES * _CPT * _CHUNK
_ACC_ROWS = 10112
_ZROWS = _ACC_ROWS // _NSUB

_sc_mesh = plsc.VectorSubcoreMesh(core_axis_name="c", subcore_axis_name="s")


@functools.partial(
    pl.kernel,
    out_type=jax.ShapeDtypeStruct((_NCORES, _ACC_ROWS, _H), jnp.float32),
    mesh=_sc_mesh,
    scratch_types=[
        pltpu.VMEM((_CPT, _CHUNK), jnp.int32),
        pltpu.VMEM((_CPT, _CHUNK), jnp.int32),
        pltpu.VMEM((_CHUNK, _H), jnp.float32),
        pltpu.VMEM((_CHUNK, _H), jnp.float32),
        pltpu.VMEM_SHARED((_ACC_ROWS, _H), jnp.float32),
        pltpu.SemaphoreType.DMA,
    ],
    compiler_params=pltpu.CompilerParams(use_tc_tiling_on_sc=False),
)
def _sc_agg(y_hbm, src_hbm, dst_hbm, zero_hbm, out_hbm, src_v, dst_v, rows0,
            rows1, acc, gsem):
    c = lax.axis_index("c")
    s = lax.axis_index("s")
    g = c * _NSUB + s
    pltpu.sync_copy(zero_hbm.at[pl.ds(s * _ZROWS, _ZROWS)],
                    acc.at[pl.ds(s * _ZROWS, _ZROWS)])
    pltpu.sync_copy(src_hbm.at[g], src_v)
    pltpu.sync_copy(dst_hbm.at[g], dst_v)
    plsc.subcore_barrier()

    def gather(j, buf):
        pltpu.async_copy(y_hbm.at[src_v.at[j]], buf, gsem)

    def gwait(buf):
        pltpu.make_async_copy(y_hbm.at[src_v.at[0]], buf, gsem).wait()

    def scatter(j, buf):
        pltpu.sync_copy(buf, acc.at[dst_v.at[j]], add=True)

    gather(0, rows0)

    def body(i, carry):
        j = 2 * i
        gwait(rows0)
        gather(j + 1, rows1)
        scatter(j, rows0)
        gwait(rows1)
        gather(j + 2, rows0)
        scatter(j + 1, rows1)
        return carry

    lax.fori_loop(0, (_CPT - 2) // 2, body, 0)
    gwait(rows0)
    gather(_CPT - 1, rows1)
    scatter(_CPT - 2, rows0)
    gwait(rows1)
    scatter(_CPT - 1, rows1)
    plsc.subcore_barrier()
    pltpu.sync_copy(acc.at[pl.ds(s * _ZROWS, _ZROWS)],
                    out_hbm.at[c, pl.ds(s * _ZROWS, _ZROWS)])


_BN = 1000


def _mm_body(x_ref, w_ref, o_ref):
    o_ref[...] = jnp.dot(x_ref[...], w_ref[...],
                         preferred_element_type=jnp.float32)


def _mm(x, w):
    n, d = x.shape
    h = w.shape[1]
    return pl.pallas_call(
        _mm_body,
        grid=(n // _BN,),
        in_specs=[
            pl.BlockSpec((_BN, d), lambda i: (i, 0)),
            pl.BlockSpec((d, h), lambda i: (0, 0)),
        ],
        out_specs=pl.BlockSpec((_BN, h), lambda i: (i, 0)),
        out_shape=jax.ShapeDtypeStruct((n, h), jnp.float32),
    )(x, w)


def _stage_b_body(y_ref, p0_ref, p1_ref, b0_ref, w1_ref, b1_ref, w2_ref, o_ref):
    h = jnp.maximum(y_ref[...] + p0_ref[...] + p1_ref[...] + b0_ref[...], 0.0)
    t = jnp.dot(h, w1_ref[...], preferred_element_type=jnp.float32) + b1_ref[...]
    t = jnp.maximum(t, 0.0)
    o_ref[...] = jnp.dot(t, w2_ref[...], preferred_element_type=jnp.float32)


def _stage_b(y1, p0, p1, b1_0, w1_1, b1_1, w2_0):
    row = lambda i: (i, 0)
    fixed = lambda i: (0, 0)
    return pl.pallas_call(
        _stage_b_body,
        grid=(_N // _BN,),
        in_specs=[
            pl.BlockSpec((_BN, _H), row),
            pl.BlockSpec((_BN, _H), row),
            pl.BlockSpec((_BN, _H), row),
            pl.BlockSpec((1, _H), fixed),
            pl.BlockSpec((_H, _H), fixed),
            pl.BlockSpec((1, _H), fixed),
            pl.BlockSpec((_H, _H), fixed),
        ],
        out_specs=pl.BlockSpec((_BN, _H), row),
        out_shape=jax.ShapeDtypeStruct((_N, _H), jnp.float32),
    )(y1, p0, p1, b1_0.reshape(1, _H), w1_1, b1_1.reshape(1, _H), w2_0)


def _stage_c_body(y_ref, p0_ref, p1_ref, b0_ref, w1_ref, b1_ref, wf_ref,
                  bf_ref, o_ref):
    h = jnp.maximum(y_ref[...] + p0_ref[...] + p1_ref[...] + b0_ref[...], 0.0)
    t = jnp.dot(h, w1_ref[...], preferred_element_type=jnp.float32) + b1_ref[...]
    logits = jnp.dot(t, wf_ref[...], preferred_element_type=jnp.float32) + bf_ref[...]
    m = jnp.max(logits, axis=1, keepdims=True)
    lse = jnp.log(jnp.sum(jnp.exp(logits - m), axis=1, keepdims=True)) + m
    o_ref[...] = logits - lse


def _stage_c(y2, p0, p1, b2_0, w2_1, b2_1, wfc, bfc):
    row = lambda i: (i, 0)
    fixed = lambda i: (0, 0)
    return pl.pallas_call(
        _stage_c_body,
        grid=(_N // _BN,),
        in_specs=[
            pl.BlockSpec((_BN, _H), row),
            pl.BlockSpec((_BN, _H), row),
            pl.BlockSpec((_BN, _H), row),
            pl.BlockSpec((1, _H), fixed),
            pl.BlockSpec((_H, _H), fixed),
            pl.BlockSpec((1, _H), fixed),
            pl.BlockSpec((_H, _C), fixed),
            pl.BlockSpec((1, _C), fixed),
        ],
        out_specs=pl.BlockSpec((_BN, _C), row),
        out_shape=jax.ShapeDtypeStruct((_N, _C), jnp.float32),
    )(y2, p0, p1, b2_0.reshape(1, _H), w2_1, b2_1.reshape(1, _H), wfc,
      bfc.reshape(1, _C))


def kernel(x, edge_index, w1_0, b1_0, w1_1, b1_1, w2_0, b2_0, w2_1, b2_1,
           wfc, bfc):
    src = edge_index[0]
    dst = edge_index[1]
    pad = _EPAD - _E
    src_p = jnp.concatenate(
        [src, jnp.zeros((pad,), jnp.int32)]).reshape(_NTILES, _CPT, _CHUNK)
    dst_p = jnp.concatenate(
        [dst, jnp.full((pad,), _N, jnp.int32)]).reshape(_NTILES, _CPT, _CHUNK)
    zeros = jnp.zeros((_ACC_ROWS, _H), jnp.float32)

    y1 = _mm(x, w1_0)
    p = _sc_agg(y1, src_p, dst_p, zeros)
    y2 = _stage_b(y1, p[0, :_N], p[1, :_N], b1_0, w1_1, b1_1, w2_0)
    q = _sc_agg(y2, src_p, dst_p, zeros)
    return _stage_c(y2, q[0, :_N], q[1, :_N], b2_0, w2_1, b2_1, wfc, bfc)


# speedup vs baseline: 1.0013x; 1.0013x over previous
import functools

import jax
import jax.numpy as jnp
from jax import lax
from jax.experimental import pallas as pl
from jax.experimental.pallas import tpu as pltpu
from jax.experimental.pallas import tpu_sc as plsc

_N = 10000
_E = 320000
_D = 128
_H = 64
_C = 64

_NCORES = 2
_NSUB = 16
_NTILES = _NCORES * _NSUB
_CHUNK = 512
_CPT = 20
_EPAD = _NTILES * _CPT * _CHUNK
_ACC_ROWS = 10112
_ZROWS = _ACC_ROWS // _NSUB

_sc_mesh = plsc.VectorSubcoreMesh(core_axis_name="c", subcore_axis_name="s")


@functools.partial(
    pl.kernel,
    out_type=jax.ShapeDtypeStruct((_NCORES, _ACC_ROWS, _H), jnp.float32),
    mesh=_sc_mesh,
    scratch_types=[
        pltpu.VMEM((_CPT, _CHUNK), jnp.int32),
        pltpu.VMEM((_CPT, _CHUNK), jnp.int32),
        pltpu.VMEM((_CHUNK, _H), jnp.float32),
        pltpu.VMEM_SHARED((_ACC_ROWS, _H), jnp.float32),
    ],
    compiler_params=pltpu.CompilerParams(use_tc_tiling_on_sc=False),
)
def _sc_agg(y_hbm, src_hbm, dst_hbm, zero_hbm, out_hbm, src_v, dst_v, rows0,
            acc):
    c = lax.axis_index("c")
    s = lax.axis_index("s")
    g = c * _NSUB + s
    pltpu.sync_copy(zero_hbm.at[pl.ds(s * _ZROWS, _ZROWS)],
                    acc.at[pl.ds(s * _ZROWS, _ZROWS)])
    pltpu.sync_copy(src_hbm.at[g], src_v)
    pltpu.sync_copy(dst_hbm.at[g], dst_v)
    plsc.subcore_barrier()

    def body(j, carry):
        pltpu.sync_copy(y_hbm.at[src_v.at[j]], rows0)
        pltpu.sync_copy(rows0, acc.at[dst_v.at[j]], add=True)
        return carry

    lax.fori_loop(0, _CPT, body, 0)
    plsc.subcore_barrier()
    pltpu.sync_copy(acc.at[pl.ds(s * _ZROWS, _ZROWS)],
                    out_hbm.at[c, pl.ds(s * _ZROWS, _ZROWS)])


_BN = 1000


def _mm_body(x_ref, w_ref, o_ref):
    o_ref[...] = jnp.dot(x_ref[...], w_ref[...],
                         preferred_element_type=jnp.float32)


def _mm(x, w):
    n, d = x.shape
    h = w.shape[1]
    return pl.pallas_call(
        _mm_body,
        grid=(n // _BN,),
        in_specs=[
            pl.BlockSpec((_BN, d), lambda i: (i, 0)),
            pl.BlockSpec((d, h), lambda i: (0, 0)),
        ],
        out_specs=pl.BlockSpec((_BN, h), lambda i: (i, 0)),
        out_shape=jax.ShapeDtypeStruct((n, h), jnp.float32),
    )(x, w)


def _stage_b_body(y_ref, p0_ref, p1_ref, b0_ref, w1_ref, b1_ref, w2_ref, o_ref):
    h = jnp.maximum(y_ref[...] + p0_ref[...] + p1_ref[...] + b0_ref[...], 0.0)
    t = jnp.dot(h, w1_ref[...], preferred_element_type=jnp.float32) + b1_ref[...]
    t = jnp.maximum(t, 0.0)
    o_ref[...] = jnp.dot(t, w2_ref[...], preferred_element_type=jnp.float32)


def _stage_b(y1, p0, p1, b1_0, w1_1, b1_1, w2_0):
    row = lambda i: (i, 0)
    fixed = lambda i: (0, 0)
    return pl.pallas_call(
        _stage_b_body,
        grid=(_N // _BN,),
        in_specs=[
            pl.BlockSpec((_BN, _H), row),
            pl.BlockSpec((_BN, _H), row),
            pl.BlockSpec((_BN, _H), row),
            pl.BlockSpec((1, _H), fixed),
            pl.BlockSpec((_H, _H), fixed),
            pl.BlockSpec((1, _H), fixed),
            pl.BlockSpec((_H, _H), fixed),
        ],
        out_specs=pl.BlockSpec((_BN, _H), row),
        out_shape=jax.ShapeDtypeStruct((_N, _H), jnp.float32),
    )(y1, p0, p1, b1_0.reshape(1, _H), w1_1, b1_1.reshape(1, _H), w2_0)


def _stage_c_body(y_ref, p0_ref, p1_ref, b0_ref, w1_ref, b1_ref, wf_ref,
                  bf_ref, o_ref):
    h = jnp.maximum(y_ref[...] + p0_ref[...] + p1_ref[...] + b0_ref[...], 0.0)
    t = jnp.dot(h, w1_ref[...], preferred_element_type=jnp.float32) + b1_ref[...]
    logits = jnp.dot(t, wf_ref[...], preferred_element_type=jnp.float32) + bf_ref[...]
    m = jnp.max(logits, axis=1, keepdims=True)
    lse = jnp.log(jnp.sum(jnp.exp(logits - m), axis=1, keepdims=True)) + m
    o_ref[...] = logits - lse


def _stage_c(y2, p0, p1, b2_0, w2_1, b2_1, wfc, bfc):
    row = lambda i: (i, 0)
    fixed = lambda i: (0, 0)
    return pl.pallas_call(
        _stage_c_body,
        grid=(_N // _BN,),
        in_specs=[
            pl.BlockSpec((_BN, _H), row),
            pl.BlockSpec((_BN, _H), row),
            pl.BlockSpec((_BN, _H), row),
            pl.BlockSpec((1, _H), fixed),
            pl.BlockSpec((_H, _H), fixed),
            pl.BlockSpec((1, _H), fixed),
            pl.BlockSpec((_H, _C), fixed),
            pl.BlockSpec((1, _C), fixed),
        ],
        out_specs=pl.BlockSpec((_BN, _C), row),
        out_shape=jax.ShapeDtypeStruct((_N, _C), jnp.float32),
    )(y2, p0, p1, b2_0.reshape(1, _H), w2_1, b2_1.reshape(1, _H), wfc,
      bfc.reshape(1, _C))


def kernel(x, edge_index, w1_0, b1_0, w1_1, b1_1, w2_0, b2_0, w2_1, b2_1,
           wfc, bfc):
    src = edge_index[0]
    dst = edge_index[1]
    pad = _EPAD - _E
    src_p = jnp.concatenate(
        [src, jnp.zeros((pad,), jnp.int32)]).reshape(_NTILES, _CPT, _CHUNK)
    dst_p = jnp.concatenate(
        [dst, jnp.full((pad,), _N, jnp.int32)]).reshape(_NTILES, _CPT, _CHUNK)
    zeros = jnp.zeros((_ACC_ROWS, _H), jnp.float32)

    y1 = _mm(x, w1_0)
    p = _sc_agg(y1, src_p, dst_p, zeros)
    y2 = _stage_b(y1, p[0, :_N], p[1, :_N], b1_0, w1_1, b1_1, w2_0)
    q = _sc_agg(y2, src_p, dst_p, zeros)
    return _stage_c(y2, q[0, :_N], q[1, :_N], b2_0, w2_1, b2_1, wfc, bfc)


# speedup vs baseline: 1.3072x; 1.3055x over previous
import functools

import jax
import jax.numpy as jnp
from jax import lax
from jax.experimental import pallas as pl
from jax.experimental.pallas import tpu as pltpu
from jax.experimental.pallas import tpu_sc as plsc

_N = 10000
_E = 320000
_D = 128
_H = 64
_C = 64

_NCORES = 2
_NSUB = 16
_NTILES = _NCORES * _NSUB
_CHUNK = 128
_CPT0 = 58
_CPT1 = 100
_CPTMAX = max(_CPT0, _CPT1)
_EPAD = _NSUB * (_CPT0 + _CPT1) * _CHUNK
_ACC_ROWS = 10112
_ZROWS = _ACC_ROWS // _NSUB

_sc_mesh = plsc.VectorSubcoreMesh(core_axis_name="c", subcore_axis_name="s")


@functools.partial(
    pl.kernel,
    out_type=jax.ShapeDtypeStruct((_NCORES, _ACC_ROWS, _H), jnp.float32),
    mesh=_sc_mesh,
    scratch_types=[
        pltpu.VMEM((_CPTMAX, _CHUNK), jnp.int32),
        pltpu.VMEM((_CPTMAX, _CHUNK), jnp.int32),
        pltpu.VMEM((_CHUNK, _H), jnp.float32),
        pltpu.VMEM_SHARED((_ACC_ROWS, _H), jnp.float32),
    ],
    compiler_params=pltpu.CompilerParams(use_tc_tiling_on_sc=False),
)
def _sc_agg(y_hbm, src_hbm, dst_hbm, zero_hbm, out_hbm, src_v, dst_v, rows0,
            acc):
    c = lax.axis_index("c")
    s = lax.axis_index("s")
    g = c * _NSUB + s
    pltpu.sync_copy(zero_hbm.at[pl.ds(s * _ZROWS, _ZROWS)],
                    acc.at[pl.ds(s * _ZROWS, _ZROWS)])
    pltpu.sync_copy(src_hbm.at[g], src_v)
    pltpu.sync_copy(dst_hbm.at[g], dst_v)
    plsc.subcore_barrier()

    def body(j, carry):
        pltpu.sync_copy(y_hbm.at[src_v.at[j]], rows0)
        pltpu.sync_copy(rows0, acc.at[dst_v.at[j]], add=True)
        return carry

    n_chunks = lax.select(c == 0, _CPT0, _CPT1)
    lax.fori_loop(0, n_chunks, body, 0)
    plsc.subcore_barrier()
    pltpu.sync_copy(acc.at[pl.ds(s * _ZROWS, _ZROWS)],
                    out_hbm.at[c, pl.ds(s * _ZROWS, _ZROWS)])


_BN = 1000


def _mm_body(x_ref, w_ref, o_ref):
    o_ref[...] = jnp.dot(x_ref[...], w_ref[...],
                         preferred_element_type=jnp.float32)


def _mm(x, w):
    n, d = x.shape
    h = w.shape[1]
    return pl.pallas_call(
        _mm_body,
        grid=(n // _BN,),
        in_specs=[
            pl.BlockSpec((_BN, d), lambda i: (i, 0)),
            pl.BlockSpec((d, h), lambda i: (0, 0)),
        ],
        out_specs=pl.BlockSpec((_BN, h), lambda i: (i, 0)),
        out_shape=jax.ShapeDtypeStruct((n, h), jnp.float32),
    )(x, w)


def _stage_b_body(y_ref, p0_ref, p1_ref, b0_ref, w1_ref, b1_ref, w2_ref, o_ref):
    h = jnp.maximum(y_ref[...] + p0_ref[...] + p1_ref[...] + b0_ref[...], 0.0)
    t = jnp.dot(h, w1_ref[...], preferred_element_type=jnp.float32) + b1_ref[...]
    t = jnp.maximum(t, 0.0)
    o_ref[...] = jnp.dot(t, w2_ref[...], preferred_element_type=jnp.float32)


def _stage_b(y1, p0, p1, b1_0, w1_1, b1_1, w2_0):
    row = lambda i: (i, 0)
    fixed = lambda i: (0, 0)
    return pl.pallas_call(
        _stage_b_body,
        grid=(_N // _BN,),
        in_specs=[
            pl.BlockSpec((_BN, _H), row),
            pl.BlockSpec((_BN, _H), row),
            pl.BlockSpec((_BN, _H), row),
            pl.BlockSpec((1, _H), fixed),
            pl.BlockSpec((_H, _H), fixed),
            pl.BlockSpec((1, _H), fixed),
            pl.BlockSpec((_H, _H), fixed),
        ],
        out_specs=pl.BlockSpec((_BN, _H), row),
        out_shape=jax.ShapeDtypeStruct((_N, _H), jnp.float32),
    )(y1, p0, p1, b1_0.reshape(1, _H), w1_1, b1_1.reshape(1, _H), w2_0)


def _stage_c_body(y_ref, p0_ref, p1_ref, b0_ref, w1_ref, b1_ref, wf_ref,
                  bf_ref, o_ref):
    h = jnp.maximum(y_ref[...] + p0_ref[...] + p1_ref[...] + b0_ref[...], 0.0)
    t = jnp.dot(h, w1_ref[...], preferred_element_type=jnp.float32) + b1_ref[...]
    logits = jnp.dot(t, wf_ref[...], preferred_element_type=jnp.float32) + bf_ref[...]
    m = jnp.max(logits, axis=1, keepdims=True)
    lse = jnp.log(jnp.sum(jnp.exp(logits - m), axis=1, keepdims=True)) + m
    o_ref[...] = logits - lse


def _stage_c(y2, p0, p1, b2_0, w2_1, b2_1, wfc, bfc):
    row = lambda i: (i, 0)
    fixed = lambda i: (0, 0)
    return pl.pallas_call(
        _stage_c_body,
        grid=(_N // _BN,),
        in_specs=[
            pl.BlockSpec((_BN, _H), row),
            pl.BlockSpec((_BN, _H), row),
            pl.BlockSpec((_BN, _H), row),
            pl.BlockSpec((1, _H), fixed),
            pl.BlockSpec((_H, _H), fixed),
            pl.BlockSpec((1, _H), fixed),
            pl.BlockSpec((_H, _C), fixed),
            pl.BlockSpec((1, _C), fixed),
        ],
        out_specs=pl.BlockSpec((_BN, _C), row),
        out_shape=jax.ShapeDtypeStruct((_N, _C), jnp.float32),
    )(y2, p0, p1, b2_0.reshape(1, _H), w2_1, b2_1.reshape(1, _H), wfc,
      bfc.reshape(1, _C))


def kernel(x, edge_index, w1_0, b1_0, w1_1, b1_1, w2_0, b2_0, w2_1, b2_1,
           wfc, bfc):
    src = edge_index[0]
    dst = edge_index[1]
    pad = _EPAD - _E

    def _ragged(e, fill):
        flat = jnp.concatenate([e, jnp.full((pad,), fill, jnp.int32)])
        n0 = _NSUB * _CPT0 * _CHUNK
        part0 = flat[:n0].reshape(_NSUB, _CPT0, _CHUNK)
        part0 = jnp.pad(part0, ((0, 0), (0, _CPTMAX - _CPT0), (0, 0)))
        part1 = flat[n0:].reshape(_NSUB, _CPT1, _CHUNK)
        part1 = jnp.pad(part1, ((0, 0), (0, _CPTMAX - _CPT1), (0, 0)))
        return jnp.concatenate([part0, part1], axis=0)

    src_p = _ragged(src, 0)
    dst_p = _ragged(dst, _N)
    zeros = jnp.zeros((_ACC_ROWS, _H), jnp.float32)

    y1 = _mm(x, w1_0)
    p = _sc_agg(y1, src_p, dst_p, zeros)
    y2 = _stage_b(y1, p[0, :_N], p[1, :_N], b1_0, w1_1, b1_1, w2_0)
    q = _sc_agg(y2, src_p, dst_p, zeros)
    return _stage_c(y2, q[0, :_N], q[1, :_N], b2_0, w2_1, b2_1, wfc, bfc)


# speedup vs baseline: 1.5950x; 1.2202x over previous
import functools

import jax
import jax.numpy as jnp
from jax import lax
from jax.experimental import pallas as pl
from jax.experimental.pallas import tpu as pltpu
from jax.experimental.pallas import tpu_sc as plsc

_N = 10000
_E = 320000
_D = 128
_H = 64
_C = 64

_NCORES = 2
_NSUB = 16
_NTILES = _NCORES * _NSUB
_CHUNK = 128
_CPT0 = 100
_CPT1 = 58
_CPTMAX = max(_CPT0, _CPT1)
_EPAD = _NSUB * (_CPT0 + _CPT1) * _CHUNK
_ACC_ROWS = 10112
_ZROWS = _ACC_ROWS // _NSUB

_sc_mesh = plsc.VectorSubcoreMesh(core_axis_name="c", subcore_axis_name="s")


@functools.partial(
    pl.kernel,
    out_type=jax.ShapeDtypeStruct((_NCORES, _ACC_ROWS, _H), jnp.float32),
    mesh=_sc_mesh,
    scratch_types=[
        pltpu.VMEM((_CPTMAX, _CHUNK), jnp.int32),
        pltpu.VMEM((_CPTMAX, _CHUNK), jnp.int32),
        pltpu.VMEM((_CHUNK, _H), jnp.float32),
        pltpu.VMEM_SHARED((_ACC_ROWS, _H), jnp.float32),
    ],
    compiler_params=pltpu.CompilerParams(use_tc_tiling_on_sc=False),
)
def _sc_agg(y_hbm, src_hbm, dst_hbm, zero_hbm, out_hbm, src_v, dst_v, rows0,
            acc):
    c = lax.axis_index("c")
    s = lax.axis_index("s")
    g = c * _NSUB + s
    pltpu.sync_copy(zero_hbm.at[pl.ds(s * _ZROWS, _ZROWS)],
                    acc.at[pl.ds(s * _ZROWS, _ZROWS)])
    pltpu.sync_copy(src_hbm.at[g], src_v)
    pltpu.sync_copy(dst_hbm.at[g], dst_v)
    plsc.subcore_barrier()

    def body(j, carry):
        pltpu.sync_copy(y_hbm.at[src_v.at[j]], rows0)
        pltpu.sync_copy(rows0, acc.at[dst_v.at[j]], add=True)
        return carry

    n_chunks = lax.select(c == 0, _CPT0, _CPT1)
    lax.fori_loop(0, n_chunks, body, 0)
    plsc.subcore_barrier()
    pltpu.sync_copy(acc.at[pl.ds(s * _ZROWS, _ZROWS)],
                    out_hbm.at[c, pl.ds(s * _ZROWS, _ZROWS)])


_BN = 1000


def _mm_body(x_ref, w_ref, o_ref):
    o_ref[...] = jnp.dot(x_ref[...], w_ref[...],
                         preferred_element_type=jnp.float32)


def _mm(x, w):
    n, d = x.shape
    h = w.shape[1]
    return pl.pallas_call(
        _mm_body,
        grid=(n // _BN,),
        in_specs=[
            pl.BlockSpec((_BN, d), lambda i: (i, 0)),
            pl.BlockSpec((d, h), lambda i: (0, 0)),
        ],
        out_specs=pl.BlockSpec((_BN, h), lambda i: (i, 0)),
        out_shape=jax.ShapeDtypeStruct((n, h), jnp.float32),
    )(x, w)


def _stage_b_body(y_ref, p0_ref, p1_ref, b0_ref, w1_ref, b1_ref, w2_ref, o_ref):
    h = jnp.maximum(y_ref[...] + p0_ref[...] + p1_ref[...] + b0_ref[...], 0.0)
    t = jnp.dot(h, w1_ref[...], preferred_element_type=jnp.float32) + b1_ref[...]
    t = jnp.maximum(t, 0.0)
    o_ref[...] = jnp.dot(t, w2_ref[...], preferred_element_type=jnp.float32)


def _stage_b(y1, p0, p1, b1_0, w1_1, b1_1, w2_0):
    row = lambda i: (i, 0)
    fixed = lambda i: (0, 0)
    return pl.pallas_call(
        _stage_b_body,
        grid=(_N // _BN,),
        in_specs=[
            pl.BlockSpec((_BN, _H), row),
            pl.BlockSpec((_BN, _H), row),
            pl.BlockSpec((_BN, _H), row),
            pl.BlockSpec((1, _H), fixed),
            pl.BlockSpec((_H, _H), fixed),
            pl.BlockSpec((1, _H), fixed),
            pl.BlockSpec((_H, _H), fixed),
        ],
        out_specs=pl.BlockSpec((_BN, _H), row),
        out_shape=jax.ShapeDtypeStruct((_N, _H), jnp.float32),
    )(y1, p0, p1, b1_0.reshape(1, _H), w1_1, b1_1.reshape(1, _H), w2_0)


def _stage_c_body(y_ref, p0_ref, p1_ref, b0_ref, w1_ref, b1_ref, wf_ref,
                  bf_ref, o_ref):
    h = jnp.maximum(y_ref[...] + p0_ref[...] + p1_ref[...] + b0_ref[...], 0.0)
    t = jnp.dot(h, w1_ref[...], preferred_element_type=jnp.float32) + b1_ref[...]
    logits = jnp.dot(t, wf_ref[...], preferred_element_type=jnp.float32) + bf_ref[...]
    m = jnp.max(logits, axis=1, keepdims=True)
    lse = jnp.log(jnp.sum(jnp.exp(logits - m), axis=1, keepdims=True)) + m
    o_ref[...] = logits - lse


def _stage_c(y2, p0, p1, b2_0, w2_1, b2_1, wfc, bfc):
    row = lambda i: (i, 0)
    fixed = lambda i: (0, 0)
    return pl.pallas_call(
        _stage_c_body,
        grid=(_N // _BN,),
        in_specs=[
            pl.BlockSpec((_BN, _H), row),
            pl.BlockSpec((_BN, _H), row),
            pl.BlockSpec((_BN, _H), row),
            pl.BlockSpec((1, _H), fixed),
            pl.BlockSpec((_H, _H), fixed),
            pl.BlockSpec((1, _H), fixed),
            pl.BlockSpec((_H, _C), fixed),
            pl.BlockSpec((1, _C), fixed),
        ],
        out_specs=pl.BlockSpec((_BN, _C), row),
        out_shape=jax.ShapeDtypeStruct((_N, _C), jnp.float32),
    )(y2, p0, p1, b2_0.reshape(1, _H), w2_1, b2_1.reshape(1, _H), wfc,
      bfc.reshape(1, _C))


def kernel(x, edge_index, w1_0, b1_0, w1_1, b1_1, w2_0, b2_0, w2_1, b2_1,
           wfc, bfc):
    src = edge_index[0]
    dst = edge_index[1]
    pad = _EPAD - _E

    def _ragged(e, fill):
        flat = jnp.concatenate([e, jnp.full((pad,), fill, jnp.int32)])
        n0 = _NSUB * _CPT0 * _CHUNK
        part0 = flat[:n0].reshape(_NSUB, _CPT0, _CHUNK)
        part0 = jnp.pad(part0, ((0, 0), (0, _CPTMAX - _CPT0), (0, 0)))
        part1 = flat[n0:].reshape(_NSUB, _CPT1, _CHUNK)
        part1 = jnp.pad(part1, ((0, 0), (0, _CPTMAX - _CPT1), (0, 0)))
        return jnp.concatenate([part0, part1], axis=0)

    src_p = _ragged(src, 0)
    dst_p = _ragged(dst, _N)
    zeros = jnp.zeros((_ACC_ROWS, _H), jnp.float32)

    y1 = _mm(x, w1_0)
    p = _sc_agg(y1, src_p, dst_p, zeros)
    y2 = _stage_b(y1, p[0, :_N], p[1, :_N], b1_0, w1_1, b1_1, w2_0)
    q = _sc_agg(y2, src_p, dst_p, zeros)
    return _stage_c(y2, q[0, :_N], q[1, :_N], b2_0, w2_1, b2_1, wfc, bfc)


# speedup vs baseline: 2.1101x; 1.3229x over previous
import functools

import jax
import jax.numpy as jnp
from jax import lax
from jax.experimental import pallas as pl
from jax.experimental.pallas import tpu as pltpu
from jax.experimental.pallas import tpu_sc as plsc

_N = 10000
_E = 320000
_D = 128
_H = 64
_C = 64

_NCORES = 2
_NSUB = 16
_NTILES = _NCORES * _NSUB
_CHUNK = 200
_CPT0 = 63
_CPT1 = 37
_EPT0 = _CPT0 * _CHUNK
_EPT1 = _CPT1 * _CHUNK
_OFF1 = _NSUB * _EPT0
_ACC_ROWS = 10112
_ZROWS = _ACC_ROWS // _NSUB

_sc_mesh = plsc.VectorSubcoreMesh(core_axis_name="c", subcore_axis_name="s")


@functools.partial(
    pl.kernel,
    out_type=jax.ShapeDtypeStruct((_NCORES, _ACC_ROWS, _H), jnp.float32),
    mesh=_sc_mesh,
    scratch_types=[
        pltpu.VMEM((_EPT0,), jnp.int32),
        pltpu.VMEM((_EPT0,), jnp.int32),
        pltpu.VMEM((_CHUNK, _H), jnp.float32),
        pltpu.VMEM_SHARED((_ACC_ROWS, _H), jnp.float32),
    ],
    compiler_params=pltpu.CompilerParams(use_tc_tiling_on_sc=False),
)
def _sc_agg(y_hbm, edge_hbm, zero_hbm, out_hbm, src_v, dst_v, rows0, acc):
    c = lax.axis_index("c")
    s = lax.axis_index("s")
    pltpu.sync_copy(zero_hbm.at[pl.ds(s * _ZROWS, _ZROWS)],
                    acc.at[pl.ds(s * _ZROWS, _ZROWS)])

    @pl.when(c == 0)
    def _():
        off = s * _EPT0
        pltpu.sync_copy(edge_hbm.at[0, pl.ds(off, _EPT0)], src_v)
        pltpu.sync_copy(edge_hbm.at[1, pl.ds(off, _EPT0)], dst_v)

    @pl.when(c == 1)
    def _():
        off = _OFF1 + s * _EPT1
        pltpu.sync_copy(edge_hbm.at[0, pl.ds(off, _EPT1)],
                        src_v.at[pl.ds(0, _EPT1)])
        pltpu.sync_copy(edge_hbm.at[1, pl.ds(off, _EPT1)],
                        dst_v.at[pl.ds(0, _EPT1)])

    plsc.subcore_barrier()

    def body(j, carry):
        e = j * _CHUNK
        pltpu.sync_copy(y_hbm.at[src_v.at[pl.ds(e, _CHUNK)]], rows0)
        pltpu.sync_copy(rows0, acc.at[dst_v.at[pl.ds(e, _CHUNK)]], add=True)
        return carry

    n_chunks = lax.select(c == 0, _CPT0, _CPT1)
    lax.fori_loop(0, n_chunks, body, 0)
    plsc.subcore_barrier()
    pltpu.sync_copy(acc.at[pl.ds(s * _ZROWS, _ZROWS)],
                    out_hbm.at[c, pl.ds(s * _ZROWS, _ZROWS)])


_BN = 1000


def _mm_body(x_ref, w_ref, o_ref):
    o_ref[...] = jnp.dot(x_ref[...], w_ref[...],
                         preferred_element_type=jnp.float32)


def _mm(x, w):
    n, d = x.shape
    h = w.shape[1]
    return pl.pallas_call(
        _mm_body,
        grid=(n // _BN,),
        in_specs=[
            pl.BlockSpec((_BN, d), lambda i: (i, 0)),
            pl.BlockSpec((d, h), lambda i: (0, 0)),
        ],
        out_specs=pl.BlockSpec((_BN, h), lambda i: (i, 0)),
        out_shape=jax.ShapeDtypeStruct((n, h), jnp.float32),
    )(x, w)


def _stage_b_body(y_ref, p0_ref, p1_ref, b0_ref, w1_ref, b1_ref, w2_ref, o_ref):
    h = jnp.maximum(y_ref[...] + p0_ref[...] + p1_ref[...] + b0_ref[...], 0.0)
    t = jnp.dot(h, w1_ref[...], preferred_element_type=jnp.float32) + b1_ref[...]
    t = jnp.maximum(t, 0.0)
    o_ref[...] = jnp.dot(t, w2_ref[...], preferred_element_type=jnp.float32)


def _stage_b(y1, p0, p1, b1_0, w1_1, b1_1, w2_0):
    row = lambda i: (i, 0)
    fixed = lambda i: (0, 0)
    return pl.pallas_call(
        _stage_b_body,
        grid=(_N // _BN,),
        in_specs=[
            pl.BlockSpec((_BN, _H), row),
            pl.BlockSpec((_BN, _H), row),
            pl.BlockSpec((_BN, _H), row),
            pl.BlockSpec((1, _H), fixed),
            pl.BlockSpec((_H, _H), fixed),
            pl.BlockSpec((1, _H), fixed),
            pl.BlockSpec((_H, _H), fixed),
        ],
        out_specs=pl.BlockSpec((_BN, _H), row),
        out_shape=jax.ShapeDtypeStruct((_N, _H), jnp.float32),
    )(y1, p0, p1, b1_0.reshape(1, _H), w1_1, b1_1.reshape(1, _H), w2_0)


def _stage_c_body(y_ref, p0_ref, p1_ref, b0_ref, w1_ref, b1_ref, wf_ref,
                  bf_ref, o_ref):
    h = jnp.maximum(y_ref[...] + p0_ref[...] + p1_ref[...] + b0_ref[...], 0.0)
    t = jnp.dot(h, w1_ref[...], preferred_element_type=jnp.float32) + b1_ref[...]
    logits = jnp.dot(t, wf_ref[...], preferred_element_type=jnp.float32) + bf_ref[...]
    m = jnp.max(logits, axis=1, keepdims=True)
    lse = jnp.log(jnp.sum(jnp.exp(logits - m), axis=1, keepdims=True)) + m
    o_ref[...] = logits - lse


def _stage_c(y2, p0, p1, b2_0, w2_1, b2_1, wfc, bfc):
    row = lambda i: (i, 0)
    fixed = lambda i: (0, 0)
    return pl.pallas_call(
        _stage_c_body,
        grid=(_N // _BN,),
        in_specs=[
            pl.BlockSpec((_BN, _H), row),
            pl.BlockSpec((_BN, _H), row),
            pl.BlockSpec((_BN, _H), row),
            pl.BlockSpec((1, _H), fixed),
            pl.BlockSpec((_H, _H), fixed),
            pl.BlockSpec((1, _H), fixed),
            pl.BlockSpec((_H, _C), fixed),
            pl.BlockSpec((1, _C), fixed),
        ],
        out_specs=pl.BlockSpec((_BN, _C), row),
        out_shape=jax.ShapeDtypeStruct((_N, _C), jnp.float32),
    )(y2, p0, p1, b2_0.reshape(1, _H), w2_1, b2_1.reshape(1, _H), wfc,
      bfc.reshape(1, _C))


def kernel(x, edge_index, w1_0, b1_0, w1_1, b1_1, w2_0, b2_0, w2_1, b2_1,
           wfc, bfc):
    zeros = jnp.zeros((_ACC_ROWS, _H), jnp.float32)

    y1 = _mm(x, w1_0)
    p = _sc_agg(y1, edge_index, zeros)
    y2 = _stage_b(y1, p[0, :_N], p[1, :_N], b1_0, w1_1, b1_1, w2_0)
    q = _sc_agg(y2, edge_index, zeros)
    return _stage_c(y2, q[0, :_N], q[1, :_N], b2_0, w2_1, b2_1, wfc, bfc)


# speedup vs baseline: 2.4031x; 1.1389x over previous
import functools

import jax
import jax.numpy as jnp
from jax import lax
from jax.experimental import pallas as pl
from jax.experimental.pallas import tpu as pltpu
from jax.experimental.pallas import tpu_sc as plsc

_N = 10000
_E = 320000
_D = 128
_H = 64
_C = 64

_NCORES = 2
_NSUB = 16
_NTILES = _NCORES * _NSUB
_CHUNK = 200
_CPT0 = 50
_CPT1 = 50
_EPT0 = _CPT0 * _CHUNK
_EPT1 = _CPT1 * _CHUNK
_OFF1 = _NSUB * _EPT0
_ACC_ROWS = 10112
_ZROWS = _ACC_ROWS // _NSUB

_sc_mesh = plsc.VectorSubcoreMesh(core_axis_name="c", subcore_axis_name="s")


@functools.partial(
    pl.kernel,
    out_type=jax.ShapeDtypeStruct((_NCORES, _ACC_ROWS, _H), jnp.float32),
    mesh=_sc_mesh,
    scratch_types=[
        pltpu.VMEM((_EPT0,), jnp.int32),
        pltpu.VMEM((_EPT0,), jnp.int32),
        pltpu.VMEM((_CHUNK, _H), jnp.float32),
        pltpu.VMEM_SHARED((_ACC_ROWS, _H), jnp.float32),
    ],
    compiler_params=pltpu.CompilerParams(use_tc_tiling_on_sc=False),
)
def _sc_agg(y_hbm, edge_hbm, zero_hbm, out_hbm, src_v, dst_v, rows0, acc):
    c = lax.axis_index("c")
    s = lax.axis_index("s")
    pltpu.sync_copy(zero_hbm.at[pl.ds(s * _ZROWS, _ZROWS)],
                    acc.at[pl.ds(s * _ZROWS, _ZROWS)])

    @pl.when(c == 0)
    def _():
        off = s * _EPT0
        pltpu.sync_copy(edge_hbm.at[0, pl.ds(off, _EPT0)], src_v)
        pltpu.sync_copy(edge_hbm.at[1, pl.ds(off, _EPT0)], dst_v)

    @pl.when(c == 1)
    def _():
        off = _OFF1 + s * _EPT1
        pltpu.sync_copy(edge_hbm.at[0, pl.ds(off, _EPT1)],
                        src_v.at[pl.ds(0, _EPT1)])
        pltpu.sync_copy(edge_hbm.at[1, pl.ds(off, _EPT1)],
                        dst_v.at[pl.ds(0, _EPT1)])

    plsc.subcore_barrier()

    def body(j, carry):
        e = j * _CHUNK
        pltpu.sync_copy(y_hbm.at[src_v.at[pl.ds(e, _CHUNK)]], rows0)
        pltpu.sync_copy(rows0, acc.at[dst_v.at[pl.ds(e, _CHUNK)]], add=True)
        return carry

    n_chunks = lax.select(c == 0, _CPT0, _CPT1)
    lax.fori_loop(0, n_chunks, body, 0)
    plsc.subcore_barrier()
    pltpu.sync_copy(acc.at[pl.ds(s * _ZROWS, _ZROWS)],
                    out_hbm.at[c, pl.ds(s * _ZROWS, _ZROWS)])


_BN = 1000


def _mm_body(x_ref, w_ref, o_ref):
    o_ref[...] = jnp.dot(x_ref[...], w_ref[...],
                         preferred_element_type=jnp.float32)


def _mm(x, w):
    n, d = x.shape
    h = w.shape[1]
    return pl.pallas_call(
        _mm_body,
        grid=(n // _BN,),
        in_specs=[
            pl.BlockSpec((_BN, d), lambda i: (i, 0)),
            pl.BlockSpec((d, h), lambda i: (0, 0)),
        ],
        out_specs=pl.BlockSpec((_BN, h), lambda i: (i, 0)),
        out_shape=jax.ShapeDtypeStruct((n, h), jnp.float32),
    )(x, w)


def _stage_b_body(y_ref, p0_ref, p1_ref, b0_ref, w1_ref, b1_ref, w2_ref, o_ref):
    h = jnp.maximum(y_ref[...] + p0_ref[...] + p1_ref[...] + b0_ref[...], 0.0)
    t = jnp.dot(h, w1_ref[...], preferred_element_type=jnp.float32) + b1_ref[...]
    t = jnp.maximum(t, 0.0)
    o_ref[...] = jnp.dot(t, w2_ref[...], preferred_element_type=jnp.float32)


def _stage_b(y1, p0, p1, b1_0, w1_1, b1_1, w2_0):
    row = lambda i: (i, 0)
    fixed = lambda i: (0, 0)
    return pl.pallas_call(
        _stage_b_body,
        grid=(_N // _BN,),
        in_specs=[
            pl.BlockSpec((_BN, _H), row),
            pl.BlockSpec((_BN, _H), row),
            pl.BlockSpec((_BN, _H), row),
            pl.BlockSpec((1, _H), fixed),
            pl.BlockSpec((_H, _H), fixed),
            pl.BlockSpec((1, _H), fixed),
            pl.BlockSpec((_H, _H), fixed),
        ],
        out_specs=pl.BlockSpec((_BN, _H), row),
        out_shape=jax.ShapeDtypeStruct((_N, _H), jnp.float32),
    )(y1, p0, p1, b1_0.reshape(1, _H), w1_1, b1_1.reshape(1, _H), w2_0)


def _stage_c_body(y_ref, p0_ref, p1_ref, b0_ref, w1_ref, b1_ref, wf_ref,
                  bf_ref, o_ref):
    h = jnp.maximum(y_ref[...] + p0_ref[...] + p1_ref[...] + b0_ref[...], 0.0)
    t = jnp.dot(h, w1_ref[...], preferred_element_type=jnp.float32) + b1_ref[...]
    logits = jnp.dot(t, wf_ref[...], preferred_element_type=jnp.float32) + bf_ref[...]
    m = jnp.max(logits, axis=1, keepdims=True)
    lse = jnp.log(jnp.sum(jnp.exp(logits - m), axis=1, keepdims=True)) + m
    o_ref[...] = logits - lse


def _stage_c(y2, p0, p1, b2_0, w2_1, b2_1, wfc, bfc):
    row = lambda i: (i, 0)
    fixed = lambda i: (0, 0)
    return pl.pallas_call(
        _stage_c_body,
        grid=(_N // _BN,),
        in_specs=[
            pl.BlockSpec((_BN, _H), row),
            pl.BlockSpec((_BN, _H), row),
            pl.BlockSpec((_BN, _H), row),
            pl.BlockSpec((1, _H), fixed),
            pl.BlockSpec((_H, _H), fixed),
            pl.BlockSpec((1, _H), fixed),
            pl.BlockSpec((_H, _C), fixed),
            pl.BlockSpec((1, _C), fixed),
        ],
        out_specs=pl.BlockSpec((_BN, _C), row),
        out_shape=jax.ShapeDtypeStruct((_N, _C), jnp.float32),
    )(y2, p0, p1, b2_0.reshape(1, _H), w2_1, b2_1.reshape(1, _H), wfc,
      bfc.reshape(1, _C))


def kernel(x, edge_index, w1_0, b1_0, w1_1, b1_1, w2_0, b2_0, w2_1, b2_1,
           wfc, bfc):
    zeros = jnp.zeros((_ACC_ROWS, _H), jnp.float32)

    y1 = _mm(x, w1_0)
    p = _sc_agg(y1, edge_index, zeros)
    y2 = _stage_b(y1, p[0, :_N], p[1, :_N], b1_0, w1_1, b1_1, w2_0)
    q = _sc_agg(y2, edge_index, zeros)
    return _stage_c(y2, q[0, :_N], q[1, :_N], b2_0, w2_1, b2_1, wfc, bfc)


# speedup vs baseline: 2.6738x; 1.1127x over previous
import functools

import jax
import jax.numpy as jnp
from jax import lax
from jax.experimental import pallas as pl
from jax.experimental.pallas import tpu as pltpu
from jax.experimental.pallas import tpu_sc as plsc

_N = 10000
_E = 320000
_D = 128
_H = 64
_C = 64

_NCORES = 2
_NSUB = 16
_NTILES = _NCORES * _NSUB
_CHUNK = 400
_CPT0 = 25
_CPT1 = 25
_EPT0 = _CPT0 * _CHUNK
_EPT1 = _CPT1 * _CHUNK
_OFF1 = _NSUB * _EPT0
_ACC_ROWS = 10112
_ZROWS = _ACC_ROWS // _NSUB

_sc_mesh = plsc.VectorSubcoreMesh(core_axis_name="c", subcore_axis_name="s")


@functools.partial(
    pl.kernel,
    out_type=jax.ShapeDtypeStruct((_NCORES, _ACC_ROWS, _H), jnp.float32),
    mesh=_sc_mesh,
    scratch_types=[
        pltpu.VMEM((_EPT0,), jnp.int32),
        pltpu.VMEM((_EPT0,), jnp.int32),
        pltpu.VMEM((_CHUNK, _H), jnp.float32),
        pltpu.VMEM_SHARED((_ACC_ROWS, _H), jnp.float32),
    ],
    compiler_params=pltpu.CompilerParams(use_tc_tiling_on_sc=False),
)
def _sc_agg(y_hbm, edge_hbm, zero_hbm, out_hbm, src_v, dst_v, rows0, acc):
    c = lax.axis_index("c")
    s = lax.axis_index("s")
    pltpu.sync_copy(zero_hbm.at[pl.ds(s * _ZROWS, _ZROWS)],
                    acc.at[pl.ds(s * _ZROWS, _ZROWS)])

    @pl.when(c == 0)
    def _():
        off = s * _EPT0
        pltpu.sync_copy(edge_hbm.at[0, pl.ds(off, _EPT0)], src_v)
        pltpu.sync_copy(edge_hbm.at[1, pl.ds(off, _EPT0)], dst_v)

    @pl.when(c == 1)
    def _():
        off = _OFF1 + s * _EPT1
        pltpu.sync_copy(edge_hbm.at[0, pl.ds(off, _EPT1)],
                        src_v.at[pl.ds(0, _EPT1)])
        pltpu.sync_copy(edge_hbm.at[1, pl.ds(off, _EPT1)],
                        dst_v.at[pl.ds(0, _EPT1)])

    plsc.subcore_barrier()

    def body(j, carry):
        e = j * _CHUNK
        pltpu.sync_copy(y_hbm.at[src_v.at[pl.ds(e, _CHUNK)]], rows0)
        pltpu.sync_copy(rows0, acc.at[dst_v.at[pl.ds(e, _CHUNK)]], add=True)
        return carry

    n_chunks = lax.select(c == 0, _CPT0, _CPT1)
    lax.fori_loop(0, n_chunks, body, 0)
    plsc.subcore_barrier()
    pltpu.sync_copy(acc.at[pl.ds(s * _ZROWS, _ZROWS)],
                    out_hbm.at[c, pl.ds(s * _ZROWS, _ZROWS)])


_BN = 1000


def _mm_body(x_ref, w_ref, o_ref):
    o_ref[...] = jnp.dot(x_ref[...], w_ref[...],
                         preferred_element_type=jnp.float32)


def _mm(x, w):
    n, d = x.shape
    h = w.shape[1]
    return pl.pallas_call(
        _mm_body,
        grid=(n // _BN,),
        in_specs=[
            pl.BlockSpec((_BN, d), lambda i: (i, 0)),
            pl.BlockSpec((d, h), lambda i: (0, 0)),
        ],
        out_specs=pl.BlockSpec((_BN, h), lambda i: (i, 0)),
        out_shape=jax.ShapeDtypeStruct((n, h), jnp.float32),
    )(x, w)


def _stage_b_body(y_ref, p0_ref, p1_ref, b0_ref, w1_ref, b1_ref, w2_ref, o_ref):
    h = jnp.maximum(y_ref[...] + p0_ref[...] + p1_ref[...] + b0_ref[...], 0.0)
    t = jnp.dot(h, w1_ref[...], preferred_element_type=jnp.float32) + b1_ref[...]
    t = jnp.maximum(t, 0.0)
    o_ref[...] = jnp.dot(t, w2_ref[...], preferred_element_type=jnp.float32)


def _stage_b(y1, p0, p1, b1_0, w1_1, b1_1, w2_0):
    row = lambda i: (i, 0)
    fixed = lambda i: (0, 0)
    return pl.pallas_call(
        _stage_b_body,
        grid=(_N // _BN,),
        in_specs=[
            pl.BlockSpec((_BN, _H), row),
            pl.BlockSpec((_BN, _H), row),
            pl.BlockSpec((_BN, _H), row),
            pl.BlockSpec((1, _H), fixed),
            pl.BlockSpec((_H, _H), fixed),
            pl.BlockSpec((1, _H), fixed),
            pl.BlockSpec((_H, _H), fixed),
        ],
        out_specs=pl.BlockSpec((_BN, _H), row),
        out_shape=jax.ShapeDtypeStruct((_N, _H), jnp.float32),
    )(y1, p0, p1, b1_0.reshape(1, _H), w1_1, b1_1.reshape(1, _H), w2_0)


def _stage_c_body(y_ref, p0_ref, p1_ref, b0_ref, w1_ref, b1_ref, wf_ref,
                  bf_ref, o_ref):
    h = jnp.maximum(y_ref[...] + p0_ref[...] + p1_ref[...] + b0_ref[...], 0.0)
    t = jnp.dot(h, w1_ref[...], preferred_element_type=jnp.float32) + b1_ref[...]
    logits = jnp.dot(t, wf_ref[...], preferred_element_type=jnp.float32) + bf_ref[...]
    m = jnp.max(logits, axis=1, keepdims=True)
    lse = jnp.log(jnp.sum(jnp.exp(logits - m), axis=1, keepdims=True)) + m
    o_ref[...] = logits - lse


def _stage_c(y2, p0, p1, b2_0, w2_1, b2_1, wfc, bfc):
    row = lambda i: (i, 0)
    fixed = lambda i: (0, 0)
    return pl.pallas_call(
        _stage_c_body,
        grid=(_N // _BN,),
        in_specs=[
            pl.BlockSpec((_BN, _H), row),
            pl.BlockSpec((_BN, _H), row),
            pl.BlockSpec((_BN, _H), row),
            pl.BlockSpec((1, _H), fixed),
            pl.BlockSpec((_H, _H), fixed),
            pl.BlockSpec((1, _H), fixed),
            pl.BlockSpec((_H, _C), fixed),
            pl.BlockSpec((1, _C), fixed),
        ],
        out_specs=pl.BlockSpec((_BN, _C), row),
        out_shape=jax.ShapeDtypeStruct((_N, _C), jnp.float32),
    )(y2, p0, p1, b2_0.reshape(1, _H), w2_1, b2_1.reshape(1, _H), wfc,
      bfc.reshape(1, _C))


def kernel(x, edge_index, w1_0, b1_0, w1_1, b1_1, w2_0, b2_0, w2_1, b2_1,
           wfc, bfc):
    zeros = jnp.zeros((_ACC_ROWS, _H), jnp.float32)

    y1 = _mm(x, w1_0)
    p = _sc_agg(y1, edge_index, zeros)
    y2 = _stage_b(y1, p[0, :_N], p[1, :_N], b1_0, w1_1, b1_1, w2_0)
    q = _sc_agg(y2, edge_index, zeros)
    return _stage_c(y2, q[0, :_N], q[1, :_N], b2_0, w2_1, b2_1, wfc, bfc)


# speedup vs baseline: 2.7989x; 1.0468x over previous
import functools

import jax
import jax.numpy as jnp
from jax import lax
from jax.experimental import pallas as pl
from jax.experimental.pallas import tpu as pltpu
from jax.experimental.pallas import tpu_sc as plsc

_N = 10000
_E = 320000
_D = 128
_H = 64
_C = 64

_NCORES = 2
_NSUB = 16
_NTILES = _NCORES * _NSUB
_CHUNK = 1000
_CPT0 = 10
_CPT1 = 10
_EPT0 = _CPT0 * _CHUNK
_EPT1 = _CPT1 * _CHUNK
_OFF1 = _NSUB * _EPT0
_ACC_ROWS = 10112
_ZROWS = _ACC_ROWS // _NSUB

_sc_mesh = plsc.VectorSubcoreMesh(core_axis_name="c", subcore_axis_name="s")


@functools.partial(
    pl.kernel,
    out_type=jax.ShapeDtypeStruct((_NCORES, _ACC_ROWS, _H), jnp.float32),
    mesh=_sc_mesh,
    scratch_types=[
        pltpu.VMEM((_EPT0,), jnp.int32),
        pltpu.VMEM((_EPT0,), jnp.int32),
        pltpu.VMEM((_CHUNK, _H), jnp.float32),
        pltpu.VMEM_SHARED((_ACC_ROWS, _H), jnp.float32),
    ],
    compiler_params=pltpu.CompilerParams(use_tc_tiling_on_sc=False),
)
def _sc_agg(y_hbm, edge_hbm, zero_hbm, out_hbm, src_v, dst_v, rows0, acc):
    c = lax.axis_index("c")
    s = lax.axis_index("s")
    pltpu.sync_copy(zero_hbm.at[pl.ds(s * _ZROWS, _ZROWS)],
                    acc.at[pl.ds(s * _ZROWS, _ZROWS)])

    @pl.when(c == 0)
    def _():
        off = s * _EPT0
        pltpu.sync_copy(edge_hbm.at[0, pl.ds(off, _EPT0)], src_v)
        pltpu.sync_copy(edge_hbm.at[1, pl.ds(off, _EPT0)], dst_v)

    @pl.when(c == 1)
    def _():
        off = _OFF1 + s * _EPT1
        pltpu.sync_copy(edge_hbm.at[0, pl.ds(off, _EPT1)],
                        src_v.at[pl.ds(0, _EPT1)])
        pltpu.sync_copy(edge_hbm.at[1, pl.ds(off, _EPT1)],
                        dst_v.at[pl.ds(0, _EPT1)])

    plsc.subcore_barrier()

    def body(j, carry):
        e = j * _CHUNK
        pltpu.sync_copy(y_hbm.at[src_v.at[pl.ds(e, _CHUNK)]], rows0)
        pltpu.sync_copy(rows0, acc.at[dst_v.at[pl.ds(e, _CHUNK)]], add=True)
        return carry

    n_chunks = lax.select(c == 0, _CPT0, _CPT1)
    lax.fori_loop(0, n_chunks, body, 0)
    plsc.subcore_barrier()
    pltpu.sync_copy(acc.at[pl.ds(s * _ZROWS, _ZROWS)],
                    out_hbm.at[c, pl.ds(s * _ZROWS, _ZROWS)])


_BN = 1000


def _mm_body(x_ref, w_ref, o_ref):
    o_ref[...] = jnp.dot(x_ref[...], w_ref[...],
                         preferred_element_type=jnp.float32)


def _mm(x, w):
    n, d = x.shape
    h = w.shape[1]
    return pl.pallas_call(
        _mm_body,
        grid=(n // _BN,),
        in_specs=[
            pl.BlockSpec((_BN, d), lambda i: (i, 0)),
            pl.BlockSpec((d, h), lambda i: (0, 0)),
        ],
        out_specs=pl.BlockSpec((_BN, h), lambda i: (i, 0)),
        out_shape=jax.ShapeDtypeStruct((n, h), jnp.float32),
    )(x, w)


def _stage_b_body(y_ref, p0_ref, p1_ref, b0_ref, w1_ref, b1_ref, w2_ref, o_ref):
    h = jnp.maximum(y_ref[...] + p0_ref[...] + p1_ref[...] + b0_ref[...], 0.0)
    t = jnp.dot(h, w1_ref[...], preferred_element_type=jnp.float32) + b1_ref[...]
    t = jnp.maximum(t, 0.0)
    o_ref[...] = jnp.dot(t, w2_ref[...], preferred_element_type=jnp.float32)


def _stage_b(y1, p0, p1, b1_0, w1_1, b1_1, w2_0):
    row = lambda i: (i, 0)
    fixed = lambda i: (0, 0)
    return pl.pallas_call(
        _stage_b_body,
        grid=(_N // _BN,),
        in_specs=[
            pl.BlockSpec((_BN, _H), row),
            pl.BlockSpec((_BN, _H), row),
            pl.BlockSpec((_BN, _H), row),
            pl.BlockSpec((1, _H), fixed),
            pl.BlockSpec((_H, _H), fixed),
            pl.BlockSpec((1, _H), fixed),
            pl.BlockSpec((_H, _H), fixed),
        ],
        out_specs=pl.BlockSpec((_BN, _H), row),
        out_shape=jax.ShapeDtypeStruct((_N, _H), jnp.float32),
    )(y1, p0, p1, b1_0.reshape(1, _H), w1_1, b1_1.reshape(1, _H), w2_0)


def _stage_c_body(y_ref, p0_ref, p1_ref, b0_ref, w1_ref, b1_ref, wf_ref,
                  bf_ref, o_ref):
    h = jnp.maximum(y_ref[...] + p0_ref[...] + p1_ref[...] + b0_ref[...], 0.0)
    t = jnp.dot(h, w1_ref[...], preferred_element_type=jnp.float32) + b1_ref[...]
    logits = jnp.dot(t, wf_ref[...], preferred_element_type=jnp.float32) + bf_ref[...]
    m = jnp.max(logits, axis=1, keepdims=True)
    lse = jnp.log(jnp.sum(jnp.exp(logits - m), axis=1, keepdims=True)) + m
    o_ref[...] = logits - lse


def _stage_c(y2, p0, p1, b2_0, w2_1, b2_1, wfc, bfc):
    row = lambda i: (i, 0)
    fixed = lambda i: (0, 0)
    return pl.pallas_call(
        _stage_c_body,
        grid=(_N // _BN,),
        in_specs=[
            pl.BlockSpec((_BN, _H), row),
            pl.BlockSpec((_BN, _H), row),
            pl.BlockSpec((_BN, _H), row),
            pl.BlockSpec((1, _H), fixed),
            pl.BlockSpec((_H, _H), fixed),
            pl.BlockSpec((1, _H), fixed),
            pl.BlockSpec((_H, _C), fixed),
            pl.BlockSpec((1, _C), fixed),
        ],
        out_specs=pl.BlockSpec((_BN, _C), row),
        out_shape=jax.ShapeDtypeStruct((_N, _C), jnp.float32),
    )(y2, p0, p1, b2_0.reshape(1, _H), w2_1, b2_1.reshape(1, _H), wfc,
      bfc.reshape(1, _C))


def kernel(x, edge_index, w1_0, b1_0, w1_1, b1_1, w2_0, b2_0, w2_1, b2_1,
           wfc, bfc):
    zeros = jnp.zeros((_ACC_ROWS, _H), jnp.float32)

    y1 = _mm(x, w1_0)
    p = _sc_agg(y1, edge_index, zeros)
    y2 = _stage_b(y1, p[0, :_N], p[1, :_N], b1_0, w1_1, b1_1, w2_0)
    q = _sc_agg(y2, edge_index, zeros)
    return _stage_c(y2, q[0, :_N], q[1, :_N], b2_0, w2_1, b2_1, wfc, bfc)


# speedup vs baseline: 2.9656x; 1.0595x over previous
import functools

import jax
import jax.numpy as jnp
from jax import lax
from jax.experimental import pallas as pl
from jax.experimental.pallas import tpu as pltpu
from jax.experimental.pallas import tpu_sc as plsc

_N = 10000
_E = 320000
_D = 128
_H = 64
_C = 64

_NCORES = 2
_NSUB = 16
_NTILES = _NCORES * _NSUB
_CHUNK = 1000
_CPT0 = 10
_CPT1 = 10
_EPT0 = _CPT0 * _CHUNK
_EPT1 = _CPT1 * _CHUNK
_OFF1 = _NSUB * _EPT0
_ACC_ROWS = 10112
_ZROWS = _ACC_ROWS // _NSUB

_sc_mesh = plsc.VectorSubcoreMesh(core_axis_name="c", subcore_axis_name="s")


@functools.partial(
    pl.kernel,
    out_type=jax.ShapeDtypeStruct((_NCORES, _ACC_ROWS, _H), jnp.float32),
    mesh=_sc_mesh,
    scratch_types=[
        pltpu.VMEM((_EPT0,), jnp.int32),
        pltpu.VMEM((_EPT0,), jnp.int32),
        pltpu.VMEM((_CHUNK, _H), jnp.float32),
        pltpu.VMEM_SHARED((_ACC_ROWS, _H), jnp.float32),
    ],
    compiler_params=pltpu.CompilerParams(use_tc_tiling_on_sc=False),
)
def _sc_agg(y_hbm, edge_hbm, zero_hbm, out_hbm, src_v, dst_v, rows0, acc):
    c = lax.axis_index("c")
    s = lax.axis_index("s")
    pltpu.sync_copy(zero_hbm.at[pl.ds(s * _ZROWS, _ZROWS)],
                    acc.at[pl.ds(s * _ZROWS, _ZROWS)])

    @pl.when(c == 0)
    def _():
        off = s * _EPT0
        pltpu.sync_copy(edge_hbm.at[0, pl.ds(off, _EPT0)], src_v)
        pltpu.sync_copy(edge_hbm.at[1, pl.ds(off, _EPT0)], dst_v)

    @pl.when(c == 1)
    def _():
        off = _OFF1 + s * _EPT1
        pltpu.sync_copy(edge_hbm.at[0, pl.ds(off, _EPT1)],
                        src_v.at[pl.ds(0, _EPT1)])
        pltpu.sync_copy(edge_hbm.at[1, pl.ds(off, _EPT1)],
                        dst_v.at[pl.ds(0, _EPT1)])

    plsc.subcore_barrier()

    def body(j, carry):
        e = j * _CHUNK
        pltpu.sync_copy(y_hbm.at[src_v.at[pl.ds(e, _CHUNK)]], rows0)
        pltpu.sync_copy(rows0, acc.at[dst_v.at[pl.ds(e, _CHUNK)]], add=True)
        return carry

    n_chunks = lax.select(c == 0, _CPT0, _CPT1)
    lax.fori_loop(0, n_chunks, body, 0)
    plsc.subcore_barrier()
    pltpu.sync_copy(acc.at[pl.ds(s * _ZROWS, _ZROWS)],
                    out_hbm.at[c, pl.ds(s * _ZROWS, _ZROWS)])


_BN = 1000


def _mm_body(x_ref, w_ref, o_ref):
    o_ref[...] = jnp.dot(x_ref[...], w_ref[...],
                         preferred_element_type=jnp.float32)


def _mm(x, w):
    n, d = x.shape
    h = w.shape[1]
    return pl.pallas_call(
        _mm_body,
        grid=(n // _BN,),
        in_specs=[
            pl.BlockSpec((_BN, d), lambda i: (i, 0)),
            pl.BlockSpec((d, h), lambda i: (0, 0)),
        ],
        out_specs=pl.BlockSpec((_BN, h), lambda i: (i, 0)),
        out_shape=jax.ShapeDtypeStruct((n, h), jnp.float32),
    )(x, w)


def _stage_b_body(y_ref, p0_ref, p1_ref, b0_ref, w1_ref, b1_ref, w2_ref, o_ref):
    h = jnp.maximum(y_ref[...] + p0_ref[0] + p1_ref[0] + b0_ref[...], 0.0)
    t = jnp.dot(h, w1_ref[...], preferred_element_type=jnp.float32) + b1_ref[...]
    t = jnp.maximum(t, 0.0)
    o_ref[...] = jnp.dot(t, w2_ref[...], preferred_element_type=jnp.float32)


def _stage_b(y1, p, b1_0, w1_1, b1_1, w2_0):
    row = lambda i: (i, 0)
    fixed = lambda i: (0, 0)
    return pl.pallas_call(
        _stage_b_body,
        grid=(_N // _BN,),
        in_specs=[
            pl.BlockSpec((_BN, _H), row),
            pl.BlockSpec((1, _BN, _H), lambda i: (0, i, 0)),
            pl.BlockSpec((1, _BN, _H), lambda i: (1, i, 0)),
            pl.BlockSpec((1, _H), fixed),
            pl.BlockSpec((_H, _H), fixed),
            pl.BlockSpec((1, _H), fixed),
            pl.BlockSpec((_H, _H), fixed),
        ],
        out_specs=pl.BlockSpec((_BN, _H), row),
        out_shape=jax.ShapeDtypeStruct((_N, _H), jnp.float32),
    )(y1, p, p, b1_0.reshape(1, _H), w1_1, b1_1.reshape(1, _H), w2_0)


def _stage_c_body(y_ref, p0_ref, p1_ref, b0_ref, w1_ref, b1_ref, wf_ref,
                  bf_ref, o_ref):
    h = jnp.maximum(y_ref[...] + p0_ref[0] + p1_ref[0] + b0_ref[...], 0.0)
    t = jnp.dot(h, w1_ref[...], preferred_element_type=jnp.float32) + b1_ref[...]
    logits = jnp.dot(t, wf_ref[...], preferred_element_type=jnp.float32) + bf_ref[...]
    m = jnp.max(logits, axis=1, keepdims=True)
    lse = jnp.log(jnp.sum(jnp.exp(logits - m), axis=1, keepdims=True)) + m
    o_ref[...] = logits - lse


def _stage_c(y2, q, b2_0, w2_1, b2_1, wfc, bfc):
    row = lambda i: (i, 0)
    fixed = lambda i: (0, 0)
    return pl.pallas_call(
        _stage_c_body,
        grid=(_N // _BN,),
        in_specs=[
            pl.BlockSpec((_BN, _H), row),
            pl.BlockSpec((1, _BN, _H), lambda i: (0, i, 0)),
            pl.BlockSpec((1, _BN, _H), lambda i: (1, i, 0)),
            pl.BlockSpec((1, _H), fixed),
            pl.BlockSpec((_H, _H), fixed),
            pl.BlockSpec((1, _H), fixed),
            pl.BlockSpec((_H, _C), fixed),
            pl.BlockSpec((1, _C), fixed),
        ],
        out_specs=pl.BlockSpec((_BN, _C), row),
        out_shape=jax.ShapeDtypeStruct((_N, _C), jnp.float32),
    )(y2, q, q, b2_0.reshape(1, _H), w2_1, b2_1.reshape(1, _H), wfc,
      bfc.reshape(1, _C))


def kernel(x, edge_index, w1_0, b1_0, w1_1, b1_1, w2_0, b2_0, w2_1, b2_1,
           wfc, bfc):
    zeros = jnp.zeros((_ACC_ROWS, _H), jnp.float32)

    y1 = _mm(x, w1_0)
    p = _sc_agg(y1, edge_index, zeros)
    y2 = _stage_b(y1, p, b1_0, w1_1, b1_1, w2_0)
    q = _sc_agg(y2, edge_index, zeros)
    return _stage_c(y2, q, b2_0, w2_1, b2_1, wfc, bfc)


# speedup vs baseline: 3.0660x; 1.0339x over previous
import functools

import jax
import jax.numpy as jnp
from jax import lax
from jax.experimental import pallas as pl
from jax.experimental.pallas import tpu as pltpu
from jax.experimental.pallas import tpu_sc as plsc

_N = 10000
_E = 320000
_D = 128
_H = 64
_C = 64

_NCORES = 2
_NSUB = 16
_NTILES = _NCORES * _NSUB
_CHUNK = 1000
_CPT0 = 10
_CPT1 = 10
_EPT0 = _CPT0 * _CHUNK
_EPT1 = _CPT1 * _CHUNK
_OFF1 = _NSUB * _EPT0
_ACC_ROWS = 10112
_ZROWS = _ACC_ROWS // _NSUB

_sc_mesh = plsc.VectorSubcoreMesh(core_axis_name="c", subcore_axis_name="s")


@functools.partial(
    pl.kernel,
    out_type=jax.ShapeDtypeStruct((_NCORES, _ACC_ROWS, _H), jnp.float32),
    mesh=_sc_mesh,
    scratch_types=[
        pltpu.VMEM((_EPT0,), jnp.int32),
        pltpu.VMEM((_EPT0,), jnp.int32),
        pltpu.VMEM((_CHUNK, _H), jnp.float32),
        pltpu.VMEM_SHARED((_ACC_ROWS, _H), jnp.float32),
    ],
    compiler_params=pltpu.CompilerParams(use_tc_tiling_on_sc=False),
)
def _sc_agg(y_hbm, edge_hbm, zero_hbm, out_hbm, src_v, dst_v, rows0, acc):
    c = lax.axis_index("c")
    s = lax.axis_index("s")
    pltpu.sync_copy(zero_hbm.at[pl.ds(s * _ZROWS, _ZROWS)],
                    acc.at[pl.ds(s * _ZROWS, _ZROWS)])

    @pl.when(c == 0)
    def _():
        off = s * _EPT0
        pltpu.sync_copy(edge_hbm.at[0, pl.ds(off, _EPT0)], src_v)
        pltpu.sync_copy(edge_hbm.at[1, pl.ds(off, _EPT0)], dst_v)

    @pl.when(c == 1)
    def _():
        off = _OFF1 + s * _EPT1
        pltpu.sync_copy(edge_hbm.at[0, pl.ds(off, _EPT1)],
                        src_v.at[pl.ds(0, _EPT1)])
        pltpu.sync_copy(edge_hbm.at[1, pl.ds(off, _EPT1)],
                        dst_v.at[pl.ds(0, _EPT1)])

    plsc.subcore_barrier()

    def body(j, carry):
        e = j * _CHUNK
        pltpu.sync_copy(y_hbm.at[src_v.at[pl.ds(e, _CHUNK)]], rows0)
        pltpu.sync_copy(rows0, acc.at[dst_v.at[pl.ds(e, _CHUNK)]], add=True)
        return carry

    n_chunks = lax.select(c == 0, _CPT0, _CPT1)
    lax.fori_loop(0, n_chunks, body, 0)
    plsc.subcore_barrier()
    pltpu.sync_copy(acc.at[pl.ds(s * _ZROWS, _ZROWS)],
                    out_hbm.at[c, pl.ds(s * _ZROWS, _ZROWS)])


_BN = 2000


def _mm_body(x_ref, w_ref, o_ref):
    o_ref[...] = jnp.dot(x_ref[...], w_ref[...],
                         preferred_element_type=jnp.float32)


def _mm(x, w):
    n, d = x.shape
    h = w.shape[1]
    return pl.pallas_call(
        _mm_body,
        grid=(n // _BN,),
        in_specs=[
            pl.BlockSpec((_BN, d), lambda i: (i, 0)),
            pl.BlockSpec((d, h), lambda i: (0, 0)),
        ],
        out_specs=pl.BlockSpec((_BN, h), lambda i: (i, 0)),
        out_shape=jax.ShapeDtypeStruct((n, h), jnp.float32),
    )(x, w)


def _stage_b_body(y_ref, p0_ref, p1_ref, b0_ref, w1_ref, b1_ref, w2_ref, o_ref):
    h = jnp.maximum(y_ref[...] + p0_ref[0] + p1_ref[0] + b0_ref[...], 0.0)
    t = jnp.dot(h, w1_ref[...], preferred_element_type=jnp.float32) + b1_ref[...]
    t = jnp.maximum(t, 0.0)
    o_ref[...] = jnp.dot(t, w2_ref[...], preferred_element_type=jnp.float32)


def _stage_b(y1, p, b1_0, w1_1, b1_1, w2_0):
    row = lambda i: (i, 0)
    fixed = lambda i: (0, 0)
    return pl.pallas_call(
        _stage_b_body,
        grid=(_N // _BN,),
        in_specs=[
            pl.BlockSpec((_BN, _H), row),
            pl.BlockSpec((1, _BN, _H), lambda i: (0, i, 0)),
            pl.BlockSpec((1, _BN, _H), lambda i: (1, i, 0)),
            pl.BlockSpec((1, _H), fixed),
            pl.BlockSpec((_H, _H), fixed),
            pl.BlockSpec((1, _H), fixed),
            pl.BlockSpec((_H, _H), fixed),
        ],
        out_specs=pl.BlockSpec((_BN, _H), row),
        out_shape=jax.ShapeDtypeStruct((_N, _H), jnp.float32),
    )(y1, p, p, b1_0.reshape(1, _H), w1_1, b1_1.reshape(1, _H), w2_0)


def _stage_c_body(y_ref, p0_ref, p1_ref, b0_ref, w1_ref, b1_ref, wf_ref,
                  bf_ref, o_ref):
    h = jnp.maximum(y_ref[...] + p0_ref[0] + p1_ref[0] + b0_ref[...], 0.0)
    t = jnp.dot(h, w1_ref[...], preferred_element_type=jnp.float32) + b1_ref[...]
    logits = jnp.dot(t, wf_ref[...], preferred_element_type=jnp.float32) + bf_ref[...]
    m = jnp.max(logits, axis=1, keepdims=True)
    lse = jnp.log(jnp.sum(jnp.exp(logits - m), axis=1, keepdims=True)) + m
    o_ref[...] = logits - lse


def _stage_c(y2, q, b2_0, w2_1, b2_1, wfc, bfc):
    row = lambda i: (i, 0)
    fixed = lambda i: (0, 0)
    return pl.pallas_call(
        _stage_c_body,
        grid=(_N // _BN,),
        in_specs=[
            pl.BlockSpec((_BN, _H), row),
            pl.BlockSpec((1, _BN, _H), lambda i: (0, i, 0)),
            pl.BlockSpec((1, _BN, _H), lambda i: (1, i, 0)),
            pl.BlockSpec((1, _H), fixed),
            pl.BlockSpec((_H, _H), fixed),
            pl.BlockSpec((1, _H), fixed),
            pl.BlockSpec((_H, _C), fixed),
            pl.BlockSpec((1, _C), fixed),
        ],
        out_specs=pl.BlockSpec((_BN, _C), row),
        out_shape=jax.ShapeDtypeStruct((_N, _C), jnp.float32),
    )(y2, q, q, b2_0.reshape(1, _H), w2_1, b2_1.reshape(1, _H), wfc,
      bfc.reshape(1, _C))


def kernel(x, edge_index, w1_0, b1_0, w1_1, b1_1, w2_0, b2_0, w2_1, b2_1,
           wfc, bfc):
    zeros = jnp.zeros((_ACC_ROWS, _H), jnp.float32)

    y1 = _mm(x, w1_0)
    p = _sc_agg(y1, edge_index, zeros)
    y2 = _stage_b(y1, p, b1_0, w1_1, b1_1, w2_0)
    q = _sc_agg(y2, edge_index, zeros)
    return _stage_c(y2, q, b2_0, w2_1, b2_1, wfc, bfc)


# speedup vs baseline: 3.1435x; 1.0253x over previous
import functools

import jax
import jax.numpy as jnp
from jax import lax
from jax.experimental import pallas as pl
from jax.experimental.pallas import tpu as pltpu
from jax.experimental.pallas import tpu_sc as plsc

_N = 10000
_E = 320000
_D = 128
_H = 64
_C = 64

_NCORES = 2
_NSUB = 16
_NTILES = _NCORES * _NSUB
_CHUNK = 1000
_CPT0 = 10
_CPT1 = 10
_EPT0 = _CPT0 * _CHUNK
_EPT1 = _CPT1 * _CHUNK
_OFF1 = _NSUB * _EPT0
_ACC_ROWS = 10112
_ZROWS = _ACC_ROWS // _NSUB

_sc_mesh = plsc.VectorSubcoreMesh(core_axis_name="c", subcore_axis_name="s")


@functools.partial(
    pl.kernel,
    out_type=jax.ShapeDtypeStruct((_NCORES, _ACC_ROWS, _H), jnp.float32),
    mesh=_sc_mesh,
    scratch_types=[
        pltpu.VMEM((_EPT0,), jnp.int32),
        pltpu.VMEM((_EPT0,), jnp.int32),
        pltpu.VMEM((_CHUNK, _H), jnp.float32),
        pltpu.VMEM_SHARED((_ACC_ROWS, _H), jnp.float32),
    ],
    compiler_params=pltpu.CompilerParams(use_tc_tiling_on_sc=False),
)
def _sc_agg(y_hbm, edge_hbm, zero_hbm, out_hbm, src_v, dst_v, rows0, acc):
    c = lax.axis_index("c")
    s = lax.axis_index("s")
    pltpu.sync_copy(zero_hbm.at[pl.ds(s * _ZROWS, _ZROWS)],
                    acc.at[pl.ds(s * _ZROWS, _ZROWS)])

    @pl.when(c == 0)
    def _():
        off = s * _EPT0
        pltpu.sync_copy(edge_hbm.at[0, pl.ds(off, _EPT0)], src_v)
        pltpu.sync_copy(edge_hbm.at[1, pl.ds(off, _EPT0)], dst_v)

    @pl.when(c == 1)
    def _():
        off = _OFF1 + s * _EPT1
        pltpu.sync_copy(edge_hbm.at[0, pl.ds(off, _EPT1)],
                        src_v.at[pl.ds(0, _EPT1)])
        pltpu.sync_copy(edge_hbm.at[1, pl.ds(off, _EPT1)],
                        dst_v.at[pl.ds(0, _EPT1)])

    plsc.subcore_barrier()

    def body(j, carry):
        e = j * _CHUNK
        pltpu.sync_copy(y_hbm.at[src_v.at[pl.ds(e, _CHUNK)]], rows0)
        pltpu.sync_copy(rows0, acc.at[dst_v.at[pl.ds(e, _CHUNK)]], add=True)
        return carry

    n_chunks = lax.select(c == 0, _CPT0, _CPT1)
    lax.fori_loop(0, n_chunks, body, 0)
    plsc.subcore_barrier()
    pltpu.sync_copy(acc.at[pl.ds(s * _ZROWS, _ZROWS)],
                    out_hbm.at[c, pl.ds(s * _ZROWS, _ZROWS)])


_BN = 5000


def _mm_body(x_ref, w_ref, o_ref):
    o_ref[...] = jnp.dot(x_ref[...], w_ref[...],
                         preferred_element_type=jnp.float32)


def _mm(x, w):
    n, d = x.shape
    h = w.shape[1]
    return pl.pallas_call(
        _mm_body,
        grid=(n // _BN,),
        in_specs=[
            pl.BlockSpec((_BN, d), lambda i: (i, 0)),
            pl.BlockSpec((d, h), lambda i: (0, 0)),
        ],
        out_specs=pl.BlockSpec((_BN, h), lambda i: (i, 0)),
        out_shape=jax.ShapeDtypeStruct((n, h), jnp.float32),
    )(x, w)


def _stage_b_body(y_ref, p0_ref, p1_ref, b0_ref, w1_ref, b1_ref, w2_ref, o_ref):
    h = jnp.maximum(y_ref[...] + p0_ref[0] + p1_ref[0] + b0_ref[...], 0.0)
    t = jnp.dot(h, w1_ref[...], preferred_element_type=jnp.float32) + b1_ref[...]
    t = jnp.maximum(t, 0.0)
    o_ref[...] = jnp.dot(t, w2_ref[...], preferred_element_type=jnp.float32)


def _stage_b(y1, p, b1_0, w1_1, b1_1, w2_0):
    row = lambda i: (i, 0)
    fixed = lambda i: (0, 0)
    return pl.pallas_call(
        _stage_b_body,
        grid=(_N // _BN,),
        in_specs=[
            pl.BlockSpec((_BN, _H), row),
            pl.BlockSpec((1, _BN, _H), lambda i: (0, i, 0)),
            pl.BlockSpec((1, _BN, _H), lambda i: (1, i, 0)),
            pl.BlockSpec((1, _H), fixed),
            pl.BlockSpec((_H, _H), fixed),
            pl.BlockSpec((1, _H), fixed),
            pl.BlockSpec((_H, _H), fixed),
        ],
        out_specs=pl.BlockSpec((_BN, _H), row),
        out_shape=jax.ShapeDtypeStruct((_N, _H), jnp.float32),
    )(y1, p, p, b1_0.reshape(1, _H), w1_1, b1_1.reshape(1, _H), w2_0)


def _stage_c_body(y_ref, p0_ref, p1_ref, b0_ref, w1_ref, b1_ref, wf_ref,
                  bf_ref, o_ref):
    h = jnp.maximum(y_ref[...] + p0_ref[0] + p1_ref[0] + b0_ref[...], 0.0)
    t = jnp.dot(h, w1_ref[...], preferred_element_type=jnp.float32) + b1_ref[...]
    logits = jnp.dot(t, wf_ref[...], preferred_element_type=jnp.float32) + bf_ref[...]
    m = jnp.max(logits, axis=1, keepdims=True)
    lse = jnp.log(jnp.sum(jnp.exp(logits - m), axis=1, keepdims=True)) + m
    o_ref[...] = logits - lse


def _stage_c(y2, q, b2_0, w2_1, b2_1, wfc, bfc):
    row = lambda i: (i, 0)
    fixed = lambda i: (0, 0)
    return pl.pallas_call(
        _stage_c_body,
        grid=(_N // _BN,),
        in_specs=[
            pl.BlockSpec((_BN, _H), row),
            pl.BlockSpec((1, _BN, _H), lambda i: (0, i, 0)),
            pl.BlockSpec((1, _BN, _H), lambda i: (1, i, 0)),
            pl.BlockSpec((1, _H), fixed),
            pl.BlockSpec((_H, _H), fixed),
            pl.BlockSpec((1, _H), fixed),
            pl.BlockSpec((_H, _C), fixed),
            pl.BlockSpec((1, _C), fixed),
        ],
        out_specs=pl.BlockSpec((_BN, _C), row),
        out_shape=jax.ShapeDtypeStruct((_N, _C), jnp.float32),
    )(y2, q, q, b2_0.reshape(1, _H), w2_1, b2_1.reshape(1, _H), wfc,
      bfc.reshape(1, _C))


def kernel(x, edge_index, w1_0, b1_0, w1_1, b1_1, w2_0, b2_0, w2_1, b2_1,
           wfc, bfc):
    zeros = jnp.zeros((_ACC_ROWS, _H), jnp.float32)

    y1 = _mm(x, w1_0)
    p = _sc_agg(y1, edge_index, zeros)
    y2 = _stage_b(y1, p, b1_0, w1_1, b1_1, w2_0)
    q = _sc_agg(y2, edge_index, zeros)
    return _stage_c(y2, q, b2_0, w2_1, b2_1, wfc, bfc)
